# Initial kernel scaffold; baseline (speedup 1.0000x reference)
import jax
import jax.numpy as jnp
from jax.experimental import pallas as pl


def kernel(x, tables):
    raise NotImplementedError("write your pallas kernel here")


# speedup vs baseline: 2.4459x; 2.4459x over previous
import functools

import jax
import jax.numpy as jnp
from jax import lax
from jax.experimental import pallas as pl
from jax.experimental.pallas import tpu as pltpu
from jax.experimental.pallas import tpu_sc as plsc

N = 100000
F = 9
V = 137
D = 128
L = 16

NW = 32
CHUNK = 224
CPW = 14
NPAD = NW * CPW * CHUNK


def _make_sc_call():
    mesh = plsc.VectorSubcoreMesh(core_axis_name="c", subcore_axis_name="s")

    @functools.partial(
        pl.kernel,
        mesh=mesh,
        out_type=jax.ShapeDtypeStruct((NPAD, D), jnp.float32),
        scratch_types=[
            pltpu.VMEM((F * CHUNK,), jnp.int32),
            pltpu.VMEM((CHUNK, D), jnp.float32),
            pltpu.VMEM((CHUNK, D), jnp.float32),
            pltpu.SemaphoreType.DMA,
        ],
    )
    def sc_call(xt_hbm, tab_hbm, out_hbm, idx_v, rows_v, acc_v, sem):
        cid = lax.axis_index("c")
        sid = lax.axis_index("s")
        wid = sid * 2 + cid

        def chunk_body(ci, carry):
            base = (wid * CPW + ci) * CHUNK
            for f in range(F):
                pltpu.sync_copy(
                    xt_hbm.at[pl.ds(f * NPAD + base, CHUNK)],
                    idx_v.at[pl.ds(f * CHUNK, CHUNK)],
                )
            for f in range(1, F):
                for k in range(CHUNK // L):
                    sl = pl.ds(f * CHUNK + k * L, L)
                    idx_v[sl] = idx_v[sl] + f * V
            pltpu.async_copy(
                tab_hbm.at[idx_v.at[pl.ds(0, CHUNK)]], acc_v, sem
            ).wait()
            for f in range(1, F):
                pltpu.async_copy(
                    tab_hbm.at[idx_v.at[pl.ds(f * CHUNK, CHUNK)]], rows_v, sem
                ).wait()

                def acc_body(j, c):
                    for k in range(D // L):
                        sl = pl.ds(k * L, L)
                        plsc.addupdate(acc_v.at[j, sl], rows_v[j, sl])
                    return c

                lax.fori_loop(0, CHUNK, acc_body, 0)
            pltpu.sync_copy(acc_v, out_hbm.at[pl.ds(base, CHUNK)])
            return carry

        lax.fori_loop(0, CPW, chunk_body, 0)

    return sc_call


_sc_call = _make_sc_call()


@jax.jit
def kernel(x, tables):
    tab2 = tables.reshape(F * V, D)
    xt = jnp.pad(x.T, ((0, 0), (0, NPAD - N))).reshape(F * NPAD)
    out = _sc_call(xt, tab2)
    return out[:N]


# speedup vs baseline: 4.8793x; 1.9949x over previous
import functools

import jax
import jax.numpy as jnp
from jax import lax
from jax.experimental import pallas as pl
from jax.experimental.pallas import tpu as pltpu
from jax.experimental.pallas import tpu_sc as plsc

N = 100000
F = 9
V = 137
D = 128
L = 16
W = D // 2

NW = 32
CHUNK = 80
FC = F * CHUNK
CPW = 40
NPAD = NW * CPW * CHUNK


def _make_sc_call():
    mesh = plsc.VectorSubcoreMesh(core_axis_name="c", subcore_axis_name="s")

    @functools.partial(
        pl.kernel,
        mesh=mesh,
        out_type=jax.ShapeDtypeStruct((NPAD, D), jnp.float32),
        compiler_params=pltpu.CompilerParams(
            needs_layout_passes=False, use_tc_tiling_on_sc=False),
        scratch_types=[
            pltpu.VMEM((FC,), jnp.int32),
            pltpu.VMEM((FC,), jnp.int32),
            pltpu.VMEM((FC, W), jnp.int32),
            pltpu.VMEM((FC, W), jnp.int32),
            pltpu.VMEM((CHUNK, D), jnp.float32),
            pltpu.VMEM((CHUNK, D), jnp.float32),
            pltpu.SemaphoreType.DMA,
            pltpu.SemaphoreType.DMA,
        ],
    )
    def sc_call(xt_hbm, tab_hbm, out_hbm, idx_a, idx_b, rows_a, rows_b,
                out_a, out_b, sem_a, sem_b):
        cid = lax.axis_index("c")
        sid = lax.axis_index("s")
        wid = sid * 2 + cid
        g0 = wid * CPW

        def stage_idx(g, idx_v):
            pltpu.sync_copy(xt_hbm.at[pl.ds(g * FC, FC)], idx_v)
            for f in range(1, F):
                for m in range(CHUNK // L):
                    sl = pl.ds(f * CHUNK + m * L, L)
                    idx_v[sl] = idx_v[sl] + f * V

        def issue_gather(idx_v, rows_v, sem):
            pltpu.async_copy(tab_hbm.at[idx_v], rows_v, sem)

        def wait_gather(idx_v, rows_v, sem):
            pltpu.make_async_copy(tab_hbm.at[idx_v], rows_v, sem).wait()

        def accumulate(rows_v, out_v):
            @plsc.parallel_loop(0, CHUNK, unroll=2)
            def _(j):
                for k in range(W // L):
                    sl = pl.ds(k * L, L)
                    s = plsc.bitcast(rows_v[j, sl], jnp.bfloat16)
                    for f in range(1, F):
                        s = s + plsc.bitcast(rows_v[f * CHUNK + j, sl],
                                             jnp.bfloat16)
                    ev, od = plsc.unpack(
                        s, format=plsc.PackFormat.INTERLEAVED,
                        preferred_element_type=jnp.float32)
                    out_v[j, pl.ds(2 * k * L, L)] = ev
                    out_v[j, pl.ds((2 * k + 1) * L, L)] = od

        def store_out(g, out_v):
            pltpu.sync_copy(out_v, out_hbm.at[pl.ds(g * CHUNK, CHUNK)])

        stage_idx(g0, idx_a)
        issue_gather(idx_a, rows_a, sem_a)

        def body2(i, carry):
            ga = g0 + 2 * i
            stage_idx(ga + 1, idx_b)
            issue_gather(idx_b, rows_b, sem_b)
            wait_gather(idx_a, rows_a, sem_a)
            accumulate(rows_a, out_a)

            @pl.when(i < CPW // 2 - 1)
            def _():
                stage_idx(ga + 2, idx_a)
                issue_gather(idx_a, rows_a, sem_a)

            store_out(ga, out_a)
            wait_gather(idx_b, rows_b, sem_b)
            accumulate(rows_b, out_b)
            store_out(ga + 1, out_b)
            return carry

        lax.fori_loop(0, CPW // 2, body2, 0)

    return sc_call


_sc_call = _make_sc_call()


@jax.jit
def kernel(x, tables):
    tb = tables.reshape(F * V, D).astype(jnp.bfloat16)
    tb = tb.reshape(F * V, D // 32, 2, L).transpose(0, 1, 3, 2)
    tabw = jax.lax.bitcast_convert_type(
        tb.reshape(F * V, W, 2), jnp.int32)

    xt = jnp.pad(x.T, ((0, 0), (0, NPAD - N)))
    xt = xt.reshape(F, NW * CPW, CHUNK).transpose(1, 0, 2).reshape(-1)

    out = _sc_call(xt, tabw)
    return out[:N]


# speedup vs baseline: 4.8871x; 1.0016x over previous
import functools

import jax
import jax.numpy as jnp
from jax import lax
from jax.experimental import pallas as pl
from jax.experimental.pallas import tpu as pltpu
from jax.experimental.pallas import tpu_sc as plsc

N = 100000
F = 9
V = 137
D = 128
L = 16
W = D // 2

NW = 32
CHUNK = 80
FC = F * CHUNK
CPW = 40
NPAD = NW * CPW * CHUNK


def _make_sc_call():
    mesh = plsc.VectorSubcoreMesh(core_axis_name="c", subcore_axis_name="s")

    @functools.partial(
        pl.kernel,
        mesh=mesh,
        out_type=jax.ShapeDtypeStruct((NPAD, D), jnp.float32),
        compiler_params=pltpu.CompilerParams(
            needs_layout_passes=False, use_tc_tiling_on_sc=False),
        scratch_types=[
            pltpu.VMEM((FC,), jnp.int32),
            pltpu.VMEM((FC,), jnp.int32),
            pltpu.VMEM((FC, W), jnp.int32),
            pltpu.VMEM((FC, W), jnp.int32),
            pltpu.VMEM((CHUNK, D), jnp.float32),
            pltpu.VMEM((CHUNK, D), jnp.float32),
            pltpu.SemaphoreType.DMA,
            pltpu.SemaphoreType.DMA,
        ],
    )
    def sc_call(xt_hbm, tab_hbm, out_hbm, idx_a, idx_b, rows_a, rows_b,
                out_a, out_b, sem_a, sem_b):
        cid = lax.axis_index("c")
        sid = lax.axis_index("s")
        wid = sid * 2 + cid
        g0 = wid * CPW

        def stage_idx(g, idx_v):
            pltpu.sync_copy(xt_hbm.at[pl.ds(g * FC, FC)], idx_v)
            for f in range(1, F):
                for m in range(CHUNK // L):
                    sl = pl.ds(f * CHUNK + m * L, L)
                    idx_v[sl] = idx_v[sl] + f * V

        def issue_gather(idx_v, rows_v, sem):
            pltpu.async_copy(tab_hbm.at[idx_v], rows_v, sem)

        def wait_gather(idx_v, rows_v, sem):
            pltpu.make_async_copy(tab_hbm.at[idx_v], rows_v, sem).wait()

        def accumulate(rows_v, out_v):
            @plsc.parallel_loop(0, CHUNK, unroll=4)
            def _(j):
                for k in range(W // L):
                    sl = pl.ds(k * L, L)
                    ts = [plsc.bitcast(rows_v[f * CHUNK + j, sl],
                                       jnp.bfloat16) for f in range(F)]
                    while len(ts) > 1:
                        nxt = [ts[i] + ts[i + 1]
                               for i in range(0, len(ts) - 1, 2)]
                        if len(ts) % 2:
                            nxt.append(ts[-1])
                        ts = nxt
                    s = ts[0]
                    ev, od = plsc.unpack(
                        s, format=plsc.PackFormat.INTERLEAVED,
                        preferred_element_type=jnp.float32)
                    out_v[j, pl.ds(2 * k * L, L)] = ev
                    out_v[j, pl.ds((2 * k + 1) * L, L)] = od

        def store_out(g, out_v):
            pltpu.sync_copy(out_v, out_hbm.at[pl.ds(g * CHUNK, CHUNK)])

        stage_idx(g0, idx_a)
        issue_gather(idx_a, rows_a, sem_a)

        def body2(i, carry):
            ga = g0 + 2 * i
            stage_idx(ga + 1, idx_b)
            issue_gather(idx_b, rows_b, sem_b)
            wait_gather(idx_a, rows_a, sem_a)
            accumulate(rows_a, out_a)

            @pl.when(i < CPW // 2 - 1)
            def _():
                stage_idx(ga + 2, idx_a)
                issue_gather(idx_a, rows_a, sem_a)

            store_out(ga, out_a)
            wait_gather(idx_b, rows_b, sem_b)
            accumulate(rows_b, out_b)
            store_out(ga + 1, out_b)
            return carry

        lax.fori_loop(0, CPW // 2, body2, 0)

    return sc_call


_sc_call = _make_sc_call()


@jax.jit
def kernel(x, tables):
    tb = tables.reshape(F * V, D).astype(jnp.bfloat16)
    tb = tb.reshape(F * V, D // 32, 2, L).transpose(0, 1, 3, 2)
    tabw = jax.lax.bitcast_convert_type(
        tb.reshape(F * V, W, 2), jnp.int32)

    xt = jnp.pad(x.T, ((0, 0), (0, NPAD - N)))
    xt = xt.reshape(F, NW * CPW, CHUNK).transpose(1, 0, 2).reshape(-1)

    out = _sc_call(xt, tabw)
    return out[:N]
